# baseline (speedup 1.0000x reference)
import functools
import math

import jax
import jax.numpy as jnp
from jax import lax
from jax.experimental import pallas as pl
from jax.experimental.pallas import tpu as pltpu
from jax.experimental.pallas import tpu_sc as plsc

S, D, E, K, H = 2048, 1024, 16, 2, 512
BLK = 128
NBR = 48
NSH = S // BLK
NBLK = NBR + NSH
PROUT = NBR * BLK
PTOT = PROUT + S


def _router_body(x_ref, wa_ref, dest_ref, be_ref, p0_ref, p1_ref, nv_ref):
    x = x_ref[...]
    aff = jax.nn.sigmoid(jnp.dot(x, wa_ref[...], preferred_element_type=jnp.float32))
    iota_e = lax.broadcasted_iota(jnp.int32, (S, E), 1)
    m0 = jnp.max(aff, axis=1, keepdims=True)
    i0 = jnp.min(jnp.where(aff == m0, iota_e, E), axis=1, keepdims=True)
    aff2 = jnp.where(iota_e == i0, -jnp.inf, aff)
    m1 = jnp.max(aff2, axis=1, keepdims=True)
    i1 = jnp.min(jnp.where(aff2 == m1, iota_e, E), axis=1, keepdims=True)
    ssum = m0 + m1
    p0_ref[...] = m0 / ssum
    p1_ref[...] = m1 / ssum

    oh0 = (iota_e == i0).astype(jnp.float32)
    oh1 = (iota_e == i1).astype(jnp.float32)
    ohs = jnp.concatenate([oh0, oh1], axis=0)
    c = ohs
    sh = 1
    while sh < 2 * S:
        c = c + jnp.concatenate([jnp.zeros((sh, E), jnp.float32), c[: 2 * S - sh]], axis=0)
        sh *= 2
    rank = jnp.sum((c - ohs) * ohs, axis=1, keepdims=True)
    counts = c[2 * S - 1 : 2 * S, :]
    nblk = jnp.floor((counts + (BLK - 1)) * (1.0 / BLK))
    padc = nblk * BLK
    iota_r = lax.broadcasted_iota(jnp.int32, (E, E), 0)
    iota_c = lax.broadcasted_iota(jnp.int32, (E, E), 1)
    lt = (iota_r < iota_c).astype(jnp.float32)
    poff = jnp.dot(padc, lt, preferred_element_type=jnp.float32)
    poff_tok = jnp.sum(ohs * poff, axis=1, keepdims=True)
    dest_ref[...] = (poff_tok + rank).astype(jnp.int32)

    blk_start = poff * (1.0 / BLK)
    i_iota = lax.broadcasted_iota(jnp.int32, (NBLK, 1), 0).astype(jnp.float32)
    ind = (i_iota >= blk_start).astype(jnp.float32)
    ber = jnp.sum(ind, axis=1, keepdims=True) - 1.0
    ber = jnp.clip(ber, 0.0, float(E - 1))
    be = jnp.where(i_iota < NBR, ber, float(E))
    be_ref[...] = be.astype(jnp.int32)
    nv_ref[...] = jnp.sum(nblk, axis=1, keepdims=True).astype(jnp.int32)


def _scatter_body(dest_ref, row_ref):
    def body(j, carry):
        d = dest_ref[j]
        row_ref[d] = j & (S - 1)
        return carry

    lax.fori_loop(0, 2 * S, body, 0)


def _make_sc_gather(nrows, chunk, tmax):
    info = plsc.get_sparse_core_info()
    nc, ns = info.num_cores, info.num_subcores
    nw = nc * ns
    per_w = nrows // nw
    assert nrows % nw == 0 and per_w % chunk == 0

    @functools.partial(
        pl.kernel,
        mesh=plsc.VectorSubcoreMesh(core_axis_name="c", subcore_axis_name="s"),
        out_type=jax.ShapeDtypeStruct((nrows, D), jnp.float32),
        scratch_types=[
            pltpu.VMEM((chunk,), jnp.int32),
            pltpu.VMEM((chunk, D), jnp.float32),
            pltpu.SemaphoreType.DMA,
        ],
    )
    def gather_k(table_hbm, idx_hbm, out_hbm, idx_v, rows_v, sem):
        wid = lax.axis_index("s") * nc + lax.axis_index("c")
        base = wid * per_w
        for c in range(per_w // chunk):
            off = pl.multiple_of(base + c * chunk, chunk)
            pltpu.sync_copy(idx_hbm.at[pl.ds(off, chunk)], idx_v)
            for j in range(chunk // 16):
                sl = pl.ds(j * 16, 16)
                idx_v[sl] = jnp.clip(idx_v[sl], 0, tmax)
            pltpu.async_copy(table_hbm.at[idx_v], rows_v, sem).wait()
            pltpu.sync_copy(rows_v, out_hbm.at[pl.ds(off, chunk)])

    return gather_k


def _ffn_body(be_ref, nv_ref, xg_ref, x_ref, w1_ref, b1_ref, w2_ref, b2_ref, y_ref):
    i = pl.program_id(0)
    live = jnp.logical_or(i >= NBR, i < nv_ref[0, 0])

    @pl.when(live)
    def _compute():
        xv = jnp.where(i < NBR, xg_ref[...], x_ref[...]).astype(jnp.bfloat16)
        t = jnp.dot(xv, w1_ref[0], preferred_element_type=jnp.float32) + b1_ref[0]
        u = t[:, :H]
        g = 0.5 * u * (1.0 + lax.erf(u * (1.0 / math.sqrt(2.0))))
        h = (g * t[:, H:]).astype(jnp.bfloat16)
        y_ref[...] = jnp.dot(h, w2_ref[0], preferred_element_type=jnp.float32) + b2_ref[0]


def _combine_body(dest_ref, x_ref, y_ref, p0_ref, p1_ref, o_ref, g0_ref, g1_ref):
    i = pl.program_id(0)

    def gb(r, carry):
        t = i * BLK + r
        d0 = dest_ref[t]
        d1 = dest_ref[S + t]
        g0_ref[pl.ds(r, 1), :] = y_ref[pl.ds(d0, 1), :]
        g1_ref[pl.ds(r, 1), :] = y_ref[pl.ds(d1, 1), :]
        return carry

    lax.fori_loop(0, BLK, gb, 0)
    ysh = y_ref[pl.ds(PROUT + i * BLK, BLK), :]
    o_ref[...] = (x_ref[...] + ysh
                  + p0_ref[...] * g0_ref[...] + p1_ref[...] * g1_ref[...])


def _moe(x2, Wa, WgW1, bgb1, W2c, b2c, interpret=False):
    f32 = jnp.float32
    i32 = jnp.int32
    dest, be, p0, p1, nv = pl.pallas_call(
        _router_body,
        out_shape=(
            jax.ShapeDtypeStruct((2 * S, 1), i32),
            jax.ShapeDtypeStruct((NBLK, 1), i32),
            jax.ShapeDtypeStruct((S, 1), f32),
            jax.ShapeDtypeStruct((S, 1), f32),
            jax.ShapeDtypeStruct((1, 1), i32),
        ),
        interpret=interpret,
    )(x2, Wa)

    dest1 = dest.reshape(2 * S)
    row_id = pl.pallas_call(
        _scatter_body,
        in_specs=[pl.BlockSpec(memory_space=pltpu.SMEM)],
        out_specs=pl.BlockSpec(memory_space=pltpu.SMEM),
        out_shape=jax.ShapeDtypeStruct((PROUT,), i32),
        interpret=interpret,
    )(dest1)

    xg = _make_sc_gather(PROUT, 64, S - 1)(x2, row_id)

    y = pl.pallas_call(
        _ffn_body,
        grid_spec=pltpu.PrefetchScalarGridSpec(
            num_scalar_prefetch=2,
            grid=(NBLK,),
            in_specs=[
                pl.BlockSpec((BLK, D), lambda i, be, nv: (jnp.minimum(i, NBR - 1), 0)),
                pl.BlockSpec((BLK, D), lambda i, be, nv: (jnp.maximum(i - NBR, 0), 0)),
                pl.BlockSpec((1, D, 2 * H), lambda i, be, nv: (be[i], 0, 0)),
                pl.BlockSpec((1, 1, 2 * H), lambda i, be, nv: (be[i], 0, 0)),
                pl.BlockSpec((1, H, D), lambda i, be, nv: (be[i], 0, 0)),
                pl.BlockSpec((1, 1, D), lambda i, be, nv: (be[i], 0, 0)),
            ],
            out_specs=pl.BlockSpec((BLK, D), lambda i, be, nv: (i, 0)),
        ),
        out_shape=jax.ShapeDtypeStruct((PTOT, D), f32),
        interpret=interpret,
    )(be.reshape(NBLK), nv, xg, x2, WgW1, bgb1, W2c, b2c)

    out = pl.pallas_call(
        _combine_body,
        grid_spec=pltpu.PrefetchScalarGridSpec(
            num_scalar_prefetch=1,
            grid=(NSH,),
            in_specs=[
                pl.BlockSpec((BLK, D), lambda i, dest: (i, 0)),
                pl.BlockSpec((PTOT, D), lambda i, dest: (0, 0)),
                pl.BlockSpec((BLK, 1), lambda i, dest: (i, 0)),
                pl.BlockSpec((BLK, 1), lambda i, dest: (i, 0)),
            ],
            out_specs=pl.BlockSpec((BLK, D), lambda i, dest: (i, 0)),
            scratch_shapes=[pltpu.VMEM((BLK, D), f32), pltpu.VMEM((BLK, D), f32)],
        ),
        out_shape=jax.ShapeDtypeStruct((S, D), f32),
        interpret=interpret,
    )(dest1, x2, y, p0, p1)
    return out


def kernel(x, Wa, Wg_s, bg_s, W1_s, b1_s, W2_s, b2_s, Wg_r, bg_r, W1_r, b1_r, W2_r, b2_r):
    x2 = x.reshape(S, D)
    WgW1 = jnp.concatenate(
        [jnp.concatenate([Wg_r, W1_r], axis=-1),
         jnp.concatenate([Wg_s, W1_s], axis=-1)[None]],
        axis=0).astype(jnp.bfloat16)
    bgb1 = jnp.concatenate(
        [jnp.concatenate([bg_r, b1_r], axis=-1),
         jnp.concatenate([bg_s, b1_s], axis=-1)[None]], axis=0).reshape(E + 1, 1, 2 * H)
    W2c = jnp.concatenate([W2_r, W2_s[None]], axis=0).astype(jnp.bfloat16)
    b2c = jnp.concatenate([b2_r, b2_s[None]], axis=0).reshape(E + 1, 1, D)
    out = _moe(x2, Wa, WgW1, bgb1, W2c, b2c)
    return out.reshape(1, S, D)


# speedup vs baseline: 1.4595x; 1.4595x over previous
import functools
import math

import jax
import jax.numpy as jnp
from jax import lax
from jax.experimental import pallas as pl
from jax.experimental.pallas import tpu as pltpu
from jax.experimental.pallas import tpu_sc as plsc

S, D, E, K, H = 2048, 1024, 16, 2, 512
BLK = 128
NBR = 48
NSH = S // BLK
NBLK = NBR + NSH
PROUT = NBR * BLK
PTOT = PROUT + S


def _router_body(x_ref, wa_ref, dest_ref, be_ref, p0_ref, p1_ref, nv_ref):
    x = x_ref[...]
    aff = jax.nn.sigmoid(jnp.dot(x, wa_ref[...], preferred_element_type=jnp.float32))
    iota_e = lax.broadcasted_iota(jnp.int32, (S, E), 1)
    m0 = jnp.max(aff, axis=1, keepdims=True)
    i0 = jnp.min(jnp.where(aff == m0, iota_e, E), axis=1, keepdims=True)
    aff2 = jnp.where(iota_e == i0, -jnp.inf, aff)
    m1 = jnp.max(aff2, axis=1, keepdims=True)
    i1 = jnp.min(jnp.where(aff2 == m1, iota_e, E), axis=1, keepdims=True)
    ssum = m0 + m1
    p0_ref[...] = m0 / ssum
    p1_ref[...] = m1 / ssum

    oh0 = (iota_e == i0).astype(jnp.float32)
    oh1 = (iota_e == i1).astype(jnp.float32)
    ohs = jnp.concatenate([oh0, oh1], axis=0)
    c = ohs
    sh = 1
    while sh < 2 * S:
        c = c + jnp.concatenate([jnp.zeros((sh, E), jnp.float32), c[: 2 * S - sh]], axis=0)
        sh *= 2
    rank = jnp.sum((c - ohs) * ohs, axis=1, keepdims=True)
    counts = c[2 * S - 1 : 2 * S, :]
    nblk = jnp.floor((counts + (BLK - 1)) * (1.0 / BLK))
    padc = nblk * BLK
    iota_r = lax.broadcasted_iota(jnp.int32, (E, E), 0)
    iota_c = lax.broadcasted_iota(jnp.int32, (E, E), 1)
    lt = (iota_r < iota_c).astype(jnp.float32)
    poff = jnp.dot(padc, lt, preferred_element_type=jnp.float32)
    poff_tok = jnp.sum(ohs * poff, axis=1, keepdims=True)
    dest_ref[...] = (poff_tok + rank).astype(jnp.int32)

    blk_start = poff * (1.0 / BLK)
    i_iota = lax.broadcasted_iota(jnp.int32, (NBLK, 1), 0).astype(jnp.float32)
    ind = (i_iota >= blk_start).astype(jnp.float32)
    ber = jnp.sum(ind, axis=1, keepdims=True) - 1.0
    ber = jnp.clip(ber, 0.0, float(E - 1))
    be = jnp.where(i_iota < NBR, ber, float(E))
    be_ref[...] = be.astype(jnp.int32)
    nv_ref[...] = jnp.sum(nblk, axis=1, keepdims=True).astype(jnp.int32)


def _scatter_body(dest_ref, row_ref):
    def body(j, carry):
        d = dest_ref[j]
        row_ref[d] = j & (S - 1)
        return carry

    lax.fori_loop(0, 2 * S, body, 0)


def _ffn_body(be_ref, rid_ref, nv_ref, x_ref, w1_ref, b1_ref, w2_ref, b2_ref, y_ref, xg_ref):
    i = pl.program_id(0)
    live_routed = jnp.logical_and(i < NBR, i < nv_ref[0, 0])

    @pl.when(live_routed)
    def _gather():
        def gb(r, carry):
            tok = jnp.clip(rid_ref[i * BLK + r], 0, S - 1)
            xg_ref[pl.ds(r, 1), :] = x_ref[pl.ds(tok, 1), :]
            return carry

        lax.fori_loop(0, BLK, gb, 0, unroll=8)

    @pl.when(i >= NBR)
    def _shared():
        xg_ref[...] = x_ref[pl.ds((i - NBR) * BLK, BLK), :]

    @pl.when(jnp.logical_or(live_routed, i >= NBR))
    def _compute():
        xv = xg_ref[...].astype(jnp.bfloat16)
        t = jnp.dot(xv, w1_ref[0], preferred_element_type=jnp.float32) + b1_ref[0]
        u = t[:, :H]
        g = 0.5 * u * (1.0 + lax.erf(u * (1.0 / math.sqrt(2.0))))
        h = (g * t[:, H:]).astype(jnp.bfloat16)
        y_ref[...] = jnp.dot(h, w2_ref[0], preferred_element_type=jnp.float32) + b2_ref[0]


def _combine_body(dest_ref, x_ref, y_ref, p0_ref, p1_ref, o_ref, g0_ref, g1_ref):
    i = pl.program_id(0)

    def gb(r, carry):
        t = i * BLK + r
        d0 = dest_ref[t]
        d1 = dest_ref[S + t]
        g0_ref[pl.ds(r, 1), :] = y_ref[pl.ds(d0, 1), :]
        g1_ref[pl.ds(r, 1), :] = y_ref[pl.ds(d1, 1), :]
        return carry

    lax.fori_loop(0, BLK, gb, 0, unroll=8)
    ysh = y_ref[pl.ds(PROUT + i * BLK, BLK), :]
    o_ref[...] = (x_ref[...] + ysh
                  + p0_ref[...] * g0_ref[...] + p1_ref[...] * g1_ref[...])


def _moe(x2, Wa, WgW1, bgb1, W2c, b2c, interpret=False):
    f32 = jnp.float32
    i32 = jnp.int32
    dest, be, p0, p1, nv = pl.pallas_call(
        _router_body,
        out_shape=(
            jax.ShapeDtypeStruct((2 * S, 1), i32),
            jax.ShapeDtypeStruct((NBLK, 1), i32),
            jax.ShapeDtypeStruct((S, 1), f32),
            jax.ShapeDtypeStruct((S, 1), f32),
            jax.ShapeDtypeStruct((1, 1), i32),
        ),
        interpret=interpret,
    )(x2, Wa)

    dest1 = dest.reshape(2 * S)
    row_id = pl.pallas_call(
        _scatter_body,
        in_specs=[pl.BlockSpec(memory_space=pltpu.SMEM)],
        out_specs=pl.BlockSpec(memory_space=pltpu.SMEM),
        out_shape=jax.ShapeDtypeStruct((PROUT,), i32),
        interpret=interpret,
    )(dest1)

    y = pl.pallas_call(
        _ffn_body,
        grid_spec=pltpu.PrefetchScalarGridSpec(
            num_scalar_prefetch=3,
            grid=(NBLK,),
            in_specs=[
                pl.BlockSpec((S, D), lambda i, be, rid, nv: (0, 0)),
                pl.BlockSpec((1, D, 2 * H), lambda i, be, rid, nv: (be[i], 0, 0)),
                pl.BlockSpec((1, 1, 2 * H), lambda i, be, rid, nv: (be[i], 0, 0)),
                pl.BlockSpec((1, H, D), lambda i, be, rid, nv: (be[i], 0, 0)),
                pl.BlockSpec((1, 1, D), lambda i, be, rid, nv: (be[i], 0, 0)),
            ],
            out_specs=pl.BlockSpec((BLK, D), lambda i, be, rid, nv: (i, 0)),
            scratch_shapes=[pltpu.VMEM((BLK, D), f32)],
        ),
        out_shape=jax.ShapeDtypeStruct((PTOT, D), f32),
        interpret=interpret,
    )(be.reshape(NBLK), row_id, nv, x2, WgW1, bgb1, W2c, b2c)

    out = pl.pallas_call(
        _combine_body,
        grid_spec=pltpu.PrefetchScalarGridSpec(
            num_scalar_prefetch=1,
            grid=(NSH,),
            in_specs=[
                pl.BlockSpec((BLK, D), lambda i, dest: (i, 0)),
                pl.BlockSpec((PTOT, D), lambda i, dest: (0, 0)),
                pl.BlockSpec((BLK, 1), lambda i, dest: (i, 0)),
                pl.BlockSpec((BLK, 1), lambda i, dest: (i, 0)),
            ],
            out_specs=pl.BlockSpec((BLK, D), lambda i, dest: (i, 0)),
            scratch_shapes=[pltpu.VMEM((BLK, D), f32), pltpu.VMEM((BLK, D), f32)],
        ),
        out_shape=jax.ShapeDtypeStruct((S, D), f32),
        interpret=interpret,
    )(dest1, x2, y, p0, p1)
    return out


def kernel(x, Wa, Wg_s, bg_s, W1_s, b1_s, W2_s, b2_s, Wg_r, bg_r, W1_r, b1_r, W2_r, b2_r):
    x2 = x.reshape(S, D)
    WgW1 = jnp.concatenate(
        [jnp.concatenate([Wg_r, W1_r], axis=-1),
         jnp.concatenate([Wg_s, W1_s], axis=-1)[None]],
        axis=0).astype(jnp.bfloat16)
    bgb1 = jnp.concatenate(
        [jnp.concatenate([bg_r, b1_r], axis=-1),
         jnp.concatenate([bg_s, b1_s], axis=-1)[None]], axis=0).reshape(E + 1, 1, 2 * H)
    W2c = jnp.concatenate([W2_r, W2_s[None]], axis=0).astype(jnp.bfloat16)
    b2c = jnp.concatenate([b2_r, b2_s[None]], axis=0).reshape(E + 1, 1, D)
    out = _moe(x2, Wa, WgW1, bgb1, W2c, b2c)
    return out.reshape(1, S, D)


# speedup vs baseline: 1.6676x; 1.1426x over previous
import functools
import math

import jax
import jax.numpy as jnp
from jax import lax
from jax.experimental import pallas as pl
from jax.experimental.pallas import tpu as pltpu
from jax.experimental.pallas import tpu_sc as plsc

S, D, E, K, H = 2048, 1024, 16, 2, 512
BLK = 256
NBR = 31
NSH = S // BLK
NBLK = NBR + NSH
PROUT = NBR * BLK
PTOT = PROUT + S


def _router_body(x_ref, wa_ref, dest_ref, be_ref, p0_ref, p1_ref, nv_ref):
    x = x_ref[...]
    aff = jax.nn.sigmoid(jnp.dot(x, wa_ref[...], preferred_element_type=jnp.float32))
    iota_e = lax.broadcasted_iota(jnp.int32, (S, E), 1)
    m0 = jnp.max(aff, axis=1, keepdims=True)
    i0 = jnp.min(jnp.where(aff == m0, iota_e, E), axis=1, keepdims=True)
    aff2 = jnp.where(iota_e == i0, -jnp.inf, aff)
    m1 = jnp.max(aff2, axis=1, keepdims=True)
    i1 = jnp.min(jnp.where(aff2 == m1, iota_e, E), axis=1, keepdims=True)
    ssum = m0 + m1
    p0_ref[...] = m0 / ssum
    p1_ref[...] = m1 / ssum

    oh0 = (iota_e == i0).astype(jnp.float32)
    oh1 = (iota_e == i1).astype(jnp.float32)
    ohs = jnp.concatenate([oh0, oh1], axis=0)
    c = ohs
    sh = 1
    while sh < 2 * S:
        c = c + jnp.concatenate([jnp.zeros((sh, E), jnp.float32), c[: 2 * S - sh]], axis=0)
        sh *= 2
    rank = jnp.sum((c - ohs) * ohs, axis=1, keepdims=True)
    counts = c[2 * S - 1 : 2 * S, :]
    nblk = jnp.floor((counts + (BLK - 1)) * (1.0 / BLK))
    padc = nblk * BLK
    iota_r = lax.broadcasted_iota(jnp.int32, (E, E), 0)
    iota_c = lax.broadcasted_iota(jnp.int32, (E, E), 1)
    lt = (iota_r < iota_c).astype(jnp.float32)
    poff = jnp.dot(padc, lt, preferred_element_type=jnp.float32)
    poff_tok = jnp.sum(ohs * poff, axis=1, keepdims=True)
    dest_ref[...] = (poff_tok + rank).astype(jnp.int32)

    blk_start = poff * (1.0 / BLK)
    i_iota = lax.broadcasted_iota(jnp.int32, (NBLK, 1), 0).astype(jnp.float32)
    ind = (i_iota >= blk_start).astype(jnp.float32)
    ber = jnp.sum(ind, axis=1, keepdims=True) - 1.0
    ber = jnp.clip(ber, 0.0, float(E - 1))
    be = jnp.where(i_iota < NBR, ber, float(E))
    be_ref[...] = be.astype(jnp.int32)
    nv_ref[...] = jnp.sum(nblk, axis=1, keepdims=True).astype(jnp.int32)


def _scatter_body(dest_ref, row_ref):
    def body(j, carry):
        d = dest_ref[j]
        row_ref[d] = j & (S - 1)
        return carry

    lax.fori_loop(0, 2 * S, body, 0, unroll=8)


def _ffn_body(be_ref, rid_ref, nv_ref, x_ref, w1_ref, b1_ref, w2_ref, b2_ref, y_ref, xg_ref):
    i = pl.program_id(0)
    live_routed = jnp.logical_and(i < NBR, i < nv_ref[0, 0])

    @pl.when(live_routed)
    def _gather():
        def gb(r, carry):
            tok = jnp.clip(rid_ref[i * BLK + r], 0, S - 1)
            xg_ref[pl.ds(r, 1), :] = x_ref[pl.ds(tok, 1), :]
            return carry

        lax.fori_loop(0, BLK, gb, 0, unroll=8)

    @pl.when(i >= NBR)
    def _shared():
        xg_ref[...] = x_ref[pl.ds((i - NBR) * BLK, BLK), :]

    @pl.when(jnp.logical_or(live_routed, i >= NBR))
    def _compute():
        xv = xg_ref[...].astype(jnp.bfloat16)
        t = jnp.dot(xv, w1_ref[0], preferred_element_type=jnp.float32) + b1_ref[0]
        u = t[:, :H]
        g = 0.5 * u * (1.0 + lax.erf(u * (1.0 / math.sqrt(2.0))))
        h = (g * t[:, H:]).astype(jnp.bfloat16)
        y_ref[...] = jnp.dot(h, w2_ref[0], preferred_element_type=jnp.float32) + b2_ref[0]


def _combine_body(dest_ref, x_ref, y_ref, p0_ref, p1_ref, o_ref, g0_ref, g1_ref):
    i = pl.program_id(0)

    def gb(r, carry):
        t = i * BLK + r
        d0 = dest_ref[t]
        d1 = dest_ref[S + t]
        g0_ref[pl.ds(r, 1), :] = y_ref[pl.ds(d0, 1), :]
        g1_ref[pl.ds(r, 1), :] = y_ref[pl.ds(d1, 1), :]
        return carry

    lax.fori_loop(0, BLK, gb, 0, unroll=8)
    ysh = y_ref[pl.ds(PROUT + i * BLK, BLK), :]
    o_ref[...] = (x_ref[...] + ysh
                  + p0_ref[...] * g0_ref[...] + p1_ref[...] * g1_ref[...])


def _moe(x2, Wa, WgW1, bgb1, W2c, b2c, interpret=False):
    f32 = jnp.float32
    i32 = jnp.int32
    dest, be, p0, p1, nv = pl.pallas_call(
        _router_body,
        out_shape=(
            jax.ShapeDtypeStruct((2 * S, 1), i32),
            jax.ShapeDtypeStruct((NBLK, 1), i32),
            jax.ShapeDtypeStruct((S, 1), f32),
            jax.ShapeDtypeStruct((S, 1), f32),
            jax.ShapeDtypeStruct((1, 1), i32),
        ),
        interpret=interpret,
    )(x2, Wa)

    dest1 = dest.reshape(2 * S)
    row_id = pl.pallas_call(
        _scatter_body,
        in_specs=[pl.BlockSpec(memory_space=pltpu.SMEM)],
        out_specs=pl.BlockSpec(memory_space=pltpu.SMEM),
        out_shape=jax.ShapeDtypeStruct((PROUT,), i32),
        interpret=interpret,
    )(dest1)

    y = pl.pallas_call(
        _ffn_body,
        grid_spec=pltpu.PrefetchScalarGridSpec(
            num_scalar_prefetch=3,
            grid=(NBLK,),
            in_specs=[
                pl.BlockSpec((S, D), lambda i, be, rid, nv: (0, 0)),
                pl.BlockSpec((1, D, 2 * H), lambda i, be, rid, nv: (be[i], 0, 0)),
                pl.BlockSpec((1, 1, 2 * H), lambda i, be, rid, nv: (be[i], 0, 0)),
                pl.BlockSpec((1, H, D), lambda i, be, rid, nv: (be[i], 0, 0)),
                pl.BlockSpec((1, 1, D), lambda i, be, rid, nv: (be[i], 0, 0)),
            ],
            out_specs=pl.BlockSpec((BLK, D), lambda i, be, rid, nv: (i, 0)),
            scratch_shapes=[pltpu.VMEM((BLK, D), f32)],
        ),
        out_shape=jax.ShapeDtypeStruct((PTOT, D), f32),
        interpret=interpret,
    )(be.reshape(NBLK), row_id, nv, x2, WgW1, bgb1, W2c, b2c)

    out = pl.pallas_call(
        _combine_body,
        grid_spec=pltpu.PrefetchScalarGridSpec(
            num_scalar_prefetch=1,
            grid=(NSH,),
            in_specs=[
                pl.BlockSpec((BLK, D), lambda i, dest: (i, 0)),
                pl.BlockSpec((PTOT, D), lambda i, dest: (0, 0)),
                pl.BlockSpec((BLK, 1), lambda i, dest: (i, 0)),
                pl.BlockSpec((BLK, 1), lambda i, dest: (i, 0)),
            ],
            out_specs=pl.BlockSpec((BLK, D), lambda i, dest: (i, 0)),
            scratch_shapes=[pltpu.VMEM((BLK, D), f32), pltpu.VMEM((BLK, D), f32)],
        ),
        out_shape=jax.ShapeDtypeStruct((S, D), f32),
        interpret=interpret,
    )(dest1, x2, y, p0, p1)
    return out


def kernel(x, Wa, Wg_s, bg_s, W1_s, b1_s, W2_s, b2_s, Wg_r, bg_r, W1_r, b1_r, W2_r, b2_r):
    x2 = x.reshape(S, D)
    WgW1 = jnp.concatenate(
        [jnp.concatenate([Wg_r, W1_r], axis=-1),
         jnp.concatenate([Wg_s, W1_s], axis=-1)[None]],
        axis=0).astype(jnp.bfloat16)
    bgb1 = jnp.concatenate(
        [jnp.concatenate([bg_r, b1_r], axis=-1),
         jnp.concatenate([bg_s, b1_s], axis=-1)[None]], axis=0).reshape(E + 1, 1, 2 * H)
    W2c = jnp.concatenate([W2_r, W2_s[None]], axis=0).astype(jnp.bfloat16)
    b2c = jnp.concatenate([b2_r, b2_s[None]], axis=0).reshape(E + 1, 1, D)
    out = _moe(x2, Wa, WgW1, bgb1, W2c, b2c)
    return out.reshape(1, S, D)
